# baseline (speedup 1.0000x reference)
import functools

import jax
import jax.numpy as jnp
from jax.experimental import pallas as pl
from jax.experimental.pallas import tpu as pltpu


def _group(t, n, max_lanes=512):
    for g in range(t, 0, -1):
        if t % g == 0 and g * n <= max_lanes and (g * n) % 128 == 0:
            return g
    return t


def _im2col(xp_ref, s, KW, N, GN):
    taps = [xp_ref[0, :, pl.ds(s * GN + dt * N, GN)] for dt in range(KW)]
    return jnp.concatenate(taps, axis=0)


def _trend_body(wm_ref, bm_ref, wt_ref, bt_ref, xp_ref, tr_ref, z1_ref, st_ref,
                *, K, KW, C, N, G):
    GN = G * N
    s = pl.program_id(1)
    im = _im2col(xp_ref, s, KW, N, GN)
    r = jnp.dot(wm_ref[...], im, preferred_element_type=jnp.float32, precision=jax.lax.Precision.HIGHEST) + bm_ref[...]
    z = jnp.zeros((C, GN), jnp.float32) + bt_ref[...]
    for k in range(1, K):
        r = jnp.concatenate(
            [jnp.dot(r[:, g * N:(g + 1) * N], tr_ref[0, g],
                     preferred_element_type=jnp.float32, precision=jax.lax.Precision.HIGHEST) for g in range(G)],
            axis=1)
        z = z + jnp.dot(wt_ref[:, (k - 1) * C:k * C], r,
                        preferred_element_type=jnp.float32, precision=jax.lax.Precision.HIGHEST)
    z1_ref[0] = z.astype(z1_ref.dtype)
    st_ref[0, 0] = jnp.concatenate(
        [jnp.sum(z, axis=1, keepdims=True),
         jnp.sum(z * z, axis=1, keepdims=True)], axis=1)


def _out_body(wf_ref, bf_ref, w1_ref, w2_ref, bp_ref, lt_ref, xp_ref, z1_ref,
              o_ref, *, K, KW, C, N, G):
    GN = G * N
    s = pl.program_id(1)
    im = _im2col(xp_ref, s, KW, N, GN)
    acc = jnp.dot(wf_ref[...], im, preferred_element_type=jnp.float32, precision=jax.lax.Precision.HIGHEST) + bf_ref[...]
    xm, x1 = acc[:C], acc[C:]
    z = (jnp.dot(w1_ref[...], z1_ref[0].astype(jnp.float32),
                 preferred_element_type=jnp.float32, precision=jax.lax.Precision.HIGHEST)
         + jnp.dot(w2_ref[:, :C], xm, preferred_element_type=jnp.float32, precision=jax.lax.Precision.HIGHEST)
         + bp_ref[...])
    for k in range(1, K):
        xk = jnp.concatenate(
            [jnp.dot(xm[:, g * N:(g + 1) * N], lt_ref[k - 1],
                     preferred_element_type=jnp.float32, precision=jax.lax.Precision.HIGHEST) for g in range(G)],
            axis=1)
        z = z + jnp.dot(w2_ref[:, k * C:(k + 1) * C], xk,
                        preferred_element_type=jnp.float32, precision=jax.lax.Precision.HIGHEST)
    filt = z[:C] + x1
    o_ref[0] = (filt * jax.nn.sigmoid(z[C:])).astype(o_ref.dtype)


def kernel(x, trend, adj, W_1, b_1, W_c1, b_c1, W_c5, b_c5, W_c7, b_c7,
           W_c9, b_c9, W_out, b_out, W_g, b_g, W_t, b_t, bn_gamma, bn_beta,
           W_f, b_f):
    with jax.default_matmul_precision("highest"):
        return _forward(x, trend, adj, W_1, b_1, W_c1, b_c1, W_c5, b_c5, W_c7,
                        b_c7, W_c9, b_c9, W_out, b_out, W_g, b_g, W_t, b_t,
                        bn_gamma, bn_beta, W_f, b_f)


def _forward(x, trend, adj, W_1, b_1, W_c1, b_c1, W_c5, b_c5, W_c7, b_c7,
             W_c9, b_c9, W_out, b_out, W_g, b_g, W_t, b_t, bn_gamma, bn_beta,
             W_f, b_f):
    B, c_in, N, T = x.shape
    C = W_1.shape[0]
    K = W_g.shape[1] // C
    KW = 9
    pad = KW // 2
    TN = T * N
    TpN = (T + 2 * pad) * N
    G = _group(T, N)
    S = T // G
    GN = G * N
    f32 = jnp.float32

    wstack = jnp.zeros((KW, c_in, 4 * C), f32)
    for q, W in enumerate((W_c1, W_c5, W_c7, W_c9)):
        kw = W.shape[-1]
        off = (KW - kw) // 2
        wstack = wstack.at[off:off + kw, :, q * C:(q + 1) * C].set(
            jnp.transpose(W[:, :, 0, :], (2, 1, 0)))
    wout = W_out[:, :, 0, 0].T
    w_m = (wstack.reshape(KW * c_in, 4 * C) @ wout).T
    b_m = (jnp.concatenate([b_c1, b_c5, b_c7, b_c9], 0) @ wout + b_out)
    w_one = jnp.zeros((KW, c_in, C), f32).at[pad].set(
        W_1[:, :, 0, 0].T).reshape(KW * c_in, C).T
    w_front = jnp.concatenate([w_m, w_one], axis=0)
    b_front = jnp.concatenate([b_m, b_1], 0).reshape(2 * C, 1)

    xt = jnp.transpose(x, (0, 1, 3, 2))
    xpad = jnp.pad(xt, ((0, 0), (0, 0), (pad, pad), (0, 0))
                   ).reshape(B, c_in, TpN)

    z1, stats = pl.pallas_call(
        functools.partial(_trend_body, K=K, KW=KW, C=C, N=N, G=G),
        out_shape=(jax.ShapeDtypeStruct((B, C, TN), x.dtype),
                   jax.ShapeDtypeStruct((B, S, C, 2), f32)),
        grid=(B, S),
        in_specs=[
            pl.BlockSpec((C, KW * c_in), lambda b, s: (0, 0)),
            pl.BlockSpec((C, 1), lambda b, s: (0, 0)),
            pl.BlockSpec((C, (K - 1) * C), lambda b, s: (0, 0)),
            pl.BlockSpec((C, 1), lambda b, s: (0, 0)),
            pl.BlockSpec((1, c_in, TpN), lambda b, s: (b, 0, 0)),
            pl.BlockSpec((1, G, N, N), lambda b, s: (b, s, 0, 0)),
        ],
        out_specs=(pl.BlockSpec((1, C, GN), lambda b, s: (b, 0, s)),
                   pl.BlockSpec((1, 1, C, 2), lambda b, s: (b, s, 0, 0))),
        compiler_params=pltpu.CompilerParams(
            dimension_semantics=("parallel", "parallel"),
            vmem_limit_bytes=32 * 1024 * 1024),
    )(w_m, b_m.reshape(C, 1), W_t[:, :, 0, 0], b_t.reshape(C, 1), xpad, trend)

    sums = stats.sum(axis=(0, 1))
    cnt = jnp.float32(B * TN)
    mean = sums[:, 0] / cnt
    var = sums[:, 1] / cnt - mean * mean
    scale = bn_gamma * jax.lax.rsqrt(var + 1e-5)
    shift = bn_beta - mean * scale
    A_f = W_f[:, :, 0, 0]
    A_f1, A_f2 = A_f[:, :C], A_f[:, C:]
    A_g = (W_g[:, :, 0, 0].reshape(2 * C, C, K)
           .transpose(0, 2, 1).reshape(2 * C, K * C))
    w1p = A_f1 * scale[None, :]
    w2p = A_f2 @ A_g
    bp = (A_f1 @ shift + A_f2 @ b_g + b_f).reshape(2 * C, 1)

    L0, L1 = jnp.eye(N, dtype=f32), adj
    lts = [L1.T]
    for _ in range(2, K):
        L2 = 2.0 * adj @ L1 - L0
        L0, L1 = L1, L2
        lts.append(L2.T)
    lt = jnp.stack(lts, axis=0)

    out = pl.pallas_call(
        functools.partial(_out_body, K=K, KW=KW, C=C, N=N, G=G),
        out_shape=jax.ShapeDtypeStruct((B, C, TN), x.dtype),
        grid=(B, S),
        in_specs=[
            pl.BlockSpec((2 * C, KW * c_in), lambda b, s: (0, 0)),
            pl.BlockSpec((2 * C, 1), lambda b, s: (0, 0)),
            pl.BlockSpec((2 * C, C), lambda b, s: (0, 0)),
            pl.BlockSpec((2 * C, K * C), lambda b, s: (0, 0)),
            pl.BlockSpec((2 * C, 1), lambda b, s: (0, 0)),
            pl.BlockSpec((K - 1, N, N), lambda b, s: (0, 0, 0)),
            pl.BlockSpec((1, c_in, TpN), lambda b, s: (b, 0, 0)),
            pl.BlockSpec((1, C, GN), lambda b, s: (b, 0, s)),
        ],
        out_specs=pl.BlockSpec((1, C, GN), lambda b, s: (b, 0, s)),
        compiler_params=pltpu.CompilerParams(
            dimension_semantics=("parallel", "parallel"),
            vmem_limit_bytes=32 * 1024 * 1024),
    )(w_front, b_front, w1p, w2p, bp, lt, xpad, z1)

    return jnp.transpose(out.reshape(B, C, T, N), (0, 1, 3, 2))


# speedup vs baseline: 1.7682x; 1.7682x over previous
import functools

import jax
import jax.numpy as jnp
from jax.experimental import pallas as pl
from jax.experimental.pallas import tpu as pltpu


def _group(t, n, max_lanes=512):
    for g in range(t, 0, -1):
        if t % g == 0 and g * n <= max_lanes and (g * n) % 128 == 0:
            return g
    return t


def _im2col(xp_ref, s, KW, N, GN):
    taps = [xp_ref[0, :, pl.ds(s * GN + dt * N, GN)] for dt in range(KW)]
    return jnp.concatenate(taps, axis=0)


def _trend_body(wm_ref, bm_ref, wt_ref, bt_ref, xp_ref, tr_ref, z1_ref, st_ref,
                *, K, KW, C, N, G):
    GN = G * N
    s = pl.program_id(1)
    im = _im2col(xp_ref, s, KW, N, GN)
    r = jnp.dot(wm_ref[...], im, preferred_element_type=jnp.float32, precision=jax.lax.Precision.DEFAULT) + bm_ref[...]
    z = jnp.zeros((C, GN), jnp.float32) + bt_ref[...]
    for k in range(1, K):
        r = jnp.concatenate(
            [jnp.dot(r[:, g * N:(g + 1) * N], tr_ref[0, g],
                     preferred_element_type=jnp.float32, precision=jax.lax.Precision.DEFAULT) for g in range(G)],
            axis=1)
        z = z + jnp.dot(wt_ref[:, (k - 1) * C:k * C], r,
                        preferred_element_type=jnp.float32, precision=jax.lax.Precision.DEFAULT)
    z1_ref[0] = z.astype(z1_ref.dtype)
    st_ref[0, 0] = jnp.concatenate(
        [jnp.sum(z, axis=1, keepdims=True),
         jnp.sum(z * z, axis=1, keepdims=True)], axis=1)


def _out_body(wf_ref, bf_ref, w1_ref, w2_ref, bp_ref, lt_ref, xp_ref, z1_ref,
              o_ref, *, K, KW, C, N, G):
    GN = G * N
    s = pl.program_id(1)
    im = _im2col(xp_ref, s, KW, N, GN)
    acc = jnp.dot(wf_ref[...], im, preferred_element_type=jnp.float32, precision=jax.lax.Precision.DEFAULT) + bf_ref[...]
    xm, x1 = acc[:C], acc[C:]
    z = (jnp.dot(w1_ref[...], z1_ref[0].astype(jnp.float32),
                 preferred_element_type=jnp.float32, precision=jax.lax.Precision.DEFAULT)
         + jnp.dot(w2_ref[:, :C], xm, preferred_element_type=jnp.float32, precision=jax.lax.Precision.DEFAULT)
         + bp_ref[...])
    for k in range(1, K):
        xk = jnp.concatenate(
            [jnp.dot(xm[:, g * N:(g + 1) * N], lt_ref[k - 1],
                     preferred_element_type=jnp.float32, precision=jax.lax.Precision.DEFAULT) for g in range(G)],
            axis=1)
        z = z + jnp.dot(w2_ref[:, k * C:(k + 1) * C], xk,
                        preferred_element_type=jnp.float32, precision=jax.lax.Precision.DEFAULT)
    filt = z[:C] + x1
    o_ref[0] = (filt * jax.nn.sigmoid(z[C:])).astype(o_ref.dtype)


def kernel(x, trend, adj, W_1, b_1, W_c1, b_c1, W_c5, b_c5, W_c7, b_c7,
           W_c9, b_c9, W_out, b_out, W_g, b_g, W_t, b_t, bn_gamma, bn_beta,
           W_f, b_f):
    with jax.default_matmul_precision("highest"):
        return _forward(x, trend, adj, W_1, b_1, W_c1, b_c1, W_c5, b_c5, W_c7,
                        b_c7, W_c9, b_c9, W_out, b_out, W_g, b_g, W_t, b_t,
                        bn_gamma, bn_beta, W_f, b_f)


def _forward(x, trend, adj, W_1, b_1, W_c1, b_c1, W_c5, b_c5, W_c7, b_c7,
             W_c9, b_c9, W_out, b_out, W_g, b_g, W_t, b_t, bn_gamma, bn_beta,
             W_f, b_f):
    B, c_in, N, T = x.shape
    C = W_1.shape[0]
    K = W_g.shape[1] // C
    KW = 9
    pad = KW // 2
    TN = T * N
    TpN = (T + 2 * pad) * N
    G = _group(T, N)
    S = T // G
    GN = G * N
    f32 = jnp.float32

    wstack = jnp.zeros((KW, c_in, 4 * C), f32)
    for q, W in enumerate((W_c1, W_c5, W_c7, W_c9)):
        kw = W.shape[-1]
        off = (KW - kw) // 2
        wstack = wstack.at[off:off + kw, :, q * C:(q + 1) * C].set(
            jnp.transpose(W[:, :, 0, :], (2, 1, 0)))
    wout = W_out[:, :, 0, 0].T
    w_m = (wstack.reshape(KW * c_in, 4 * C) @ wout).T
    b_m = (jnp.concatenate([b_c1, b_c5, b_c7, b_c9], 0) @ wout + b_out)
    w_one = jnp.zeros((KW, c_in, C), f32).at[pad].set(
        W_1[:, :, 0, 0].T).reshape(KW * c_in, C).T
    w_front = jnp.concatenate([w_m, w_one], axis=0)
    b_front = jnp.concatenate([b_m, b_1], 0).reshape(2 * C, 1)

    xt = jnp.transpose(x, (0, 1, 3, 2))
    xpad = jnp.pad(xt, ((0, 0), (0, 0), (pad, pad), (0, 0))
                   ).reshape(B, c_in, TpN)

    z1, stats = pl.pallas_call(
        functools.partial(_trend_body, K=K, KW=KW, C=C, N=N, G=G),
        out_shape=(jax.ShapeDtypeStruct((B, C, TN), x.dtype),
                   jax.ShapeDtypeStruct((B, S, C, 2), f32)),
        grid=(B, S),
        in_specs=[
            pl.BlockSpec((C, KW * c_in), lambda b, s: (0, 0)),
            pl.BlockSpec((C, 1), lambda b, s: (0, 0)),
            pl.BlockSpec((C, (K - 1) * C), lambda b, s: (0, 0)),
            pl.BlockSpec((C, 1), lambda b, s: (0, 0)),
            pl.BlockSpec((1, c_in, TpN), lambda b, s: (b, 0, 0)),
            pl.BlockSpec((1, G, N, N), lambda b, s: (b, s, 0, 0)),
        ],
        out_specs=(pl.BlockSpec((1, C, GN), lambda b, s: (b, 0, s)),
                   pl.BlockSpec((1, 1, C, 2), lambda b, s: (b, s, 0, 0))),
        compiler_params=pltpu.CompilerParams(
            dimension_semantics=("parallel", "parallel"),
            vmem_limit_bytes=32 * 1024 * 1024),
    )(w_m, b_m.reshape(C, 1), W_t[:, :, 0, 0], b_t.reshape(C, 1), xpad, trend)

    sums = stats.sum(axis=(0, 1))
    cnt = jnp.float32(B * TN)
    mean = sums[:, 0] / cnt
    var = sums[:, 1] / cnt - mean * mean
    scale = bn_gamma * jax.lax.rsqrt(var + 1e-5)
    shift = bn_beta - mean * scale
    A_f = W_f[:, :, 0, 0]
    A_f1, A_f2 = A_f[:, :C], A_f[:, C:]
    A_g = (W_g[:, :, 0, 0].reshape(2 * C, C, K)
           .transpose(0, 2, 1).reshape(2 * C, K * C))
    w1p = A_f1 * scale[None, :]
    w2p = A_f2 @ A_g
    bp = (A_f1 @ shift + A_f2 @ b_g + b_f).reshape(2 * C, 1)

    L0, L1 = jnp.eye(N, dtype=f32), adj
    lts = [L1.T]
    for _ in range(2, K):
        L2 = 2.0 * adj @ L1 - L0
        L0, L1 = L1, L2
        lts.append(L2.T)
    lt = jnp.stack(lts, axis=0)

    out = pl.pallas_call(
        functools.partial(_out_body, K=K, KW=KW, C=C, N=N, G=G),
        out_shape=jax.ShapeDtypeStruct((B, C, TN), x.dtype),
        grid=(B, S),
        in_specs=[
            pl.BlockSpec((2 * C, KW * c_in), lambda b, s: (0, 0)),
            pl.BlockSpec((2 * C, 1), lambda b, s: (0, 0)),
            pl.BlockSpec((2 * C, C), lambda b, s: (0, 0)),
            pl.BlockSpec((2 * C, K * C), lambda b, s: (0, 0)),
            pl.BlockSpec((2 * C, 1), lambda b, s: (0, 0)),
            pl.BlockSpec((K - 1, N, N), lambda b, s: (0, 0, 0)),
            pl.BlockSpec((1, c_in, TpN), lambda b, s: (b, 0, 0)),
            pl.BlockSpec((1, C, GN), lambda b, s: (b, 0, s)),
        ],
        out_specs=pl.BlockSpec((1, C, GN), lambda b, s: (b, 0, s)),
        compiler_params=pltpu.CompilerParams(
            dimension_semantics=("parallel", "parallel"),
            vmem_limit_bytes=32 * 1024 * 1024),
    )(w_front, b_front, w1p, w2p, bp, lt, xpad, z1)

    return jnp.transpose(out.reshape(B, C, T, N), (0, 1, 3, 2))
